# baseline (speedup 1.0000x reference)
import functools

import jax
import jax.numpy as jnp
from jax.experimental import pallas as pl
from jax.experimental.pallas import tpu as pltpu

_EPS = 1e-8
_CHUNK = 128
_RING = 4
_SUB = 32


def _cdiv(a, b):
    return -(-a // b)


def _neg_snr_rows(x1, x2, length, eps):
    r = x1.shape[0]
    n_chunks = _cdiv(length, _CHUNK)

    z = jnp.zeros((r, _CHUNK), jnp.float32)
    m1, m2, v11, v22, v12 = z, z, z, z, z
    for c in range(n_chunks):
        off = c * _CHUNK
        x1c = x1[:, off:off + _CHUNK]
        x2c = x2[:, off:off + _CHUNK]
        if off + _CHUNK > length:
            lane = jax.lax.broadcasted_iota(jnp.int32, (r, _CHUNK), 1)
            keep = lane < (length - off)
            x1c = jnp.where(keep, x1c, 0.0)
            x2c = jnp.where(keep, x2c, 0.0)
        m1 = m1 + x1c
        m2 = m2 + x2c
        v11 = v11 + x1c * x1c
        v22 = v22 + x2c * x2c
        v12 = v12 + x1c * x2c

    s1m = jnp.sum(m1, axis=-1, keepdims=True)
    s2m = jnp.sum(m2, axis=-1, keepdims=True)
    p11 = jnp.sum(v11, axis=-1, keepdims=True)
    p22 = jnp.sum(v22, axis=-1, keepdims=True)
    p12 = jnp.sum(v12, axis=-1, keepdims=True)

    inv_len = jnp.float32(1.0 / length)
    c11 = p11 - s1m * s1m * inv_len
    c22 = p22 - s2m * s2m * inv_len
    c12 = p12 - s1m * s2m * inv_len

    alpha = c12 / (c22 + eps)
    target = alpha * alpha * c22
    noise = c11 - 2.0 * alpha * c12 + target
    return -10.0 * jnp.log10(target / (noise + eps) + eps)


def _sub_plan(rows_per_core):
    if rows_per_core >= 2 * _SUB and rows_per_core % _SUB == 0:
        rs = [_SUB // 2, _SUB // 2] + [_SUB] * (rows_per_core // _SUB - 1)
    else:
        rs = [min(_SUB, rows_per_core)]
        while sum(rs) < rows_per_core:
            rs.append(min(_SUB, rows_per_core - sum(rs)))
    offs = [0]
    for r in rs[:-1]:
        offs.append(offs[-1] + r)
    return list(zip(offs, rs))


def _sdr_manual_kernel(s1_hbm, s2_hbm, out_ref, b1, b2, sem1, sem2, *,
                       rows_per_core, length, eps):
    core = pl.program_id(0)
    row0 = core * rows_per_core
    plan = _sub_plan(rows_per_core)
    n = len(plan)

    def start(t):
        off, r = plan[t]
        slot = t % _RING
        src_rows = pl.ds(row0 + off, r)
        pltpu.make_async_copy(
            s1_hbm.at[src_rows, :], b1.at[slot, pl.ds(0, r), :],
            sem1.at[slot]).start()
        pltpu.make_async_copy(
            s2_hbm.at[src_rows, :], b2.at[slot, pl.ds(0, r), :],
            sem2.at[slot]).start()

    def wait(t):
        _, r = plan[t]
        slot = t % _RING
        pltpu.make_async_copy(
            b1.at[slot, pl.ds(0, r), :], b1.at[slot, pl.ds(0, r), :],
            sem1.at[slot]).wait()
        pltpu.make_async_copy(
            b2.at[slot, pl.ds(0, r), :], b2.at[slot, pl.ds(0, r), :],
            sem2.at[slot]).wait()

    for t in range(min(_RING - 1, n)):
        start(t)
    for t in range(n):
        if t + _RING - 1 < n:
            start(t + _RING - 1)
        wait(t)
        off, r = plan[t]
        slot = t % _RING
        x1 = b1[slot, :r, :]
        x2 = b2[slot, :r, :]
        out_ref[pl.ds(off, r), :] = _neg_snr_rows(x1, x2, length, eps)


def _sdr_auto_kernel(s1_ref, s2_ref, out_ref, *, length, eps):
    out_ref[...] = _neg_snr_rows(s1_ref[...], s2_ref[...], length, eps)


def _auto_path(s1, s2, B, L, Lp):
    tb = 32 if B % 32 == 0 else (8 if B % 8 == 0 else B)
    n_b = _cdiv(B, tb)
    body = functools.partial(_sdr_auto_kernel, length=L, eps=_EPS)
    return pl.pallas_call(
        body,
        out_shape=jax.ShapeDtypeStruct((n_b * tb, 1), jnp.float32),
        grid=(n_b,),
        in_specs=[
            pl.BlockSpec((tb, Lp), lambda i: (i, 0)),
            pl.BlockSpec((tb, Lp), lambda i: (i, 0)),
        ],
        out_specs=pl.BlockSpec((tb, 1), lambda i: (i, 0)),
        compiler_params=pltpu.CompilerParams(
            dimension_semantics=("parallel",),
            vmem_limit_bytes=48 * 1024 * 1024,
        ),
    )(s1, s2)


def _manual_path(s1, s2, B, L):
    rows_per_core = B // 2
    body = functools.partial(
        _sdr_manual_kernel,
        rows_per_core=rows_per_core, length=L, eps=_EPS,
    )
    return pl.pallas_call(
        body,
        out_shape=jax.ShapeDtypeStruct((B, 1), jnp.float32),
        grid=(2,),
        in_specs=[
            pl.BlockSpec(memory_space=pl.ANY),
            pl.BlockSpec(memory_space=pl.ANY),
        ],
        out_specs=pl.BlockSpec((rows_per_core, 1), lambda i: (i, 0)),
        scratch_shapes=[
            pltpu.VMEM((_RING, _SUB, L), jnp.float32),
            pltpu.VMEM((_RING, _SUB, L), jnp.float32),
            pltpu.SemaphoreType.DMA((_RING,)),
            pltpu.SemaphoreType.DMA((_RING,)),
        ],
        compiler_params=pltpu.CompilerParams(
            dimension_semantics=("parallel",),
            vmem_limit_bytes=56 * 1024 * 1024,
        ),
    )(s1, s2)


def kernel(s1, s2):
    assert s1.ndim == 2 and s1.shape == s2.shape
    B, L = s1.shape
    Lp = _cdiv(L, _CHUNK) * _CHUNK
    if B % 32 == 0 and B >= 64 and L == Lp:
        neg_snr = _manual_path(s1, s2, B, L)
    else:
        neg_snr = _auto_path(s1, s2, B, L, Lp)
    return jnp.mean(neg_snr[:B])


# speedup vs baseline: 1.1432x; 1.1432x over previous
import functools

import jax
import jax.numpy as jnp
from jax.experimental import pallas as pl
from jax.experimental.pallas import tpu as pltpu

_EPS = 1e-8
_CHUNK = 128


def _cdiv(a, b):
    return -(-a // b)


def _neg_snr_rows(x1_ref, x2_ref, length, eps):
    tb = x1_ref.shape[0]
    n_chunks = _cdiv(length, _CHUNK)

    z = jnp.zeros((tb, _CHUNK), jnp.float32)
    m1, m2, v11, v22, v12 = z, z, z, z, z
    for c in range(n_chunks):
        off = c * _CHUNK
        x1 = x1_ref[:, off:off + _CHUNK]
        x2 = x2_ref[:, off:off + _CHUNK]
        if off + _CHUNK > length:
            lane = jax.lax.broadcasted_iota(jnp.int32, (tb, _CHUNK), 1)
            keep = lane < (length - off)
            x1 = jnp.where(keep, x1, 0.0)
            x2 = jnp.where(keep, x2, 0.0)
        m1 = m1 + x1
        m2 = m2 + x2
        v11 = v11 + x1 * x1
        v22 = v22 + x2 * x2
        v12 = v12 + x1 * x2

    s1m = jnp.sum(m1, axis=-1, keepdims=True)
    s2m = jnp.sum(m2, axis=-1, keepdims=True)
    p11 = jnp.sum(v11, axis=-1, keepdims=True)
    p22 = jnp.sum(v22, axis=-1, keepdims=True)
    p12 = jnp.sum(v12, axis=-1, keepdims=True)

    inv_len = jnp.float32(1.0 / length)
    c11 = p11 - s1m * s1m * inv_len
    c22 = p22 - s2m * s2m * inv_len
    c12 = p12 - s1m * s2m * inv_len

    alpha = c12 / (c22 + eps)
    target = alpha * alpha * c22
    noise = c11 - 2.0 * alpha * c12 + target
    return -10.0 * jnp.log10(target / (noise + eps) + eps)


def _sdr_kernel(s1_ref, s2_ref, out_ref, *, length, eps):
    out_ref[...] = _neg_snr_rows(s1_ref, s2_ref, length, eps)


def kernel(s1, s2):
    assert s1.ndim == 2 and s1.shape == s2.shape
    B, L = s1.shape
    Lp = _cdiv(L, _CHUNK) * _CHUNK
    tb = 32 if B % 32 == 0 else (8 if B % 8 == 0 else B)
    n_b = _cdiv(B, tb)

    body = functools.partial(_sdr_kernel, length=L, eps=_EPS)

    neg_snr = pl.pallas_call(
        body,
        out_shape=jax.ShapeDtypeStruct((n_b * tb, 1), jnp.float32),
        grid=(n_b,),
        in_specs=[
            pl.BlockSpec((tb, Lp), lambda i: (i, 0)),
            pl.BlockSpec((tb, Lp), lambda i: (i, 0)),
        ],
        out_specs=pl.BlockSpec((tb, 1), lambda i: (i, 0)),
        compiler_params=pltpu.CompilerParams(
            dimension_semantics=("parallel",),
            vmem_limit_bytes=48 * 1024 * 1024,
        ),
    )(s1, s2)

    return jnp.mean(neg_snr[:B])
